# baseline (speedup 1.0000x reference)
import functools

import jax
import jax.numpy as jnp
from jax import lax
from jax.experimental import pallas as pl
from jax.experimental.pallas import tpu as pltpu, tpu_sc as plsc

VOCAB = 1000000
EMB = 64
N_PART = 4
PART = EMB // N_PART
BATCH = 16384
HIST = 50
B = BATCH * HIST

NW = 32
B_PER_W = B // NW
CHUNK = 800
N_CHUNKS = B_PER_W // CHUNK

NBLK_FULL = VOCAB // 128
TAIL_V = NBLK_FULL * 128
Q_ROWS = VOCAB // 8
BLK_BASE = NBLK_FULL // NW
BLK_REM = NBLK_FULL - BLK_BASE * NW


def _make_relayout():
    mesh = plsc.VectorSubcoreMesh(core_axis_name="c", subcore_axis_name="s")
    out_t = jax.ShapeDtypeStruct((Q_ROWS, 128), jnp.float32)

    @functools.partial(
        pl.kernel,
        mesh=mesh,
        out_type=(out_t, out_t, out_t, out_t),
        scratch_types=[
            [pltpu.VMEM((16, 128), jnp.float32) for _ in range(8)],
            [pltpu.VMEM((16, 128), jnp.float32) for _ in range(8)],
            pltpu.VMEM((8, 128), jnp.float32),
            pltpu.SemaphoreType.DMA,
            pltpu.SemaphoreType.DMA,
            pltpu.SemaphoreType.DMA,
            pltpu.SemaphoreType.DMA,
        ],
        compiler_params=pltpu.CompilerParams(
            use_tc_tiling_on_sc=True, needs_layout_passes=False,
            disable_bounds_checks=True),
    )
    def relayout_kernel(t0, t1, t2, t3, tq0, tq1, tq2, tq3,
                        q0, q1, q2, q3, inbufs, obufs, tstage,
                        isem_a, isem_b, osem_a, osem_b):
        wid = lax.axis_index("s") * 2 + lax.axis_index("c")
        ts = (t0, t1, t2, t3)
        tqs = (tq0, tq1, tq2, tq3)
        qs = (q0, q1, q2, q3)
        start = wid * BLK_BASE + jnp.minimum(wid, BLK_REM)
        nb = jnp.where(wid < BLK_REM, BLK_BASE + 1, BLK_BASE)
        lanes = lax.iota(jnp.int32, 16)
        isems = (isem_a, isem_b)
        osems = (osem_a, osem_b)

        def stage_in(k, p):
            v0 = (start + k) * 128
            for t in range(4):
                pltpu.async_copy(
                    ts[t].at[:, pl.ds(v0, 128)], inbufs[4 * p + t], isems[p])

        def wait_in(k, p):
            v0 = (start + k) * 128
            for t in range(4):
                pltpu.make_async_copy(
                    ts[t].at[:, pl.ds(v0, 128)], inbufs[4 * p + t],
                    isems[p]).wait()

        def drain_out(k, p):
            blk = start + k
            for t in range(4):
                pltpu.make_async_copy(
                    obufs[4 * p + t], qs[t].at[pl.ds(blk * 16, 16)],
                    osems[p]).wait()

        def extract(p):
            @plsc.parallel_loop(0, 16, unroll=4)
            def group_body(g):
                c8 = g * 8
                rows = jnp.broadcast_to(g, (16,))
                for dc in range(8):
                    vcol = jnp.broadcast_to(c8 + dc, (16,))
                    cols = dc * PART + lanes
                    for t in range(4):
                        row = plsc.load_gather(inbufs[4 * p + t], [lanes, vcol])
                        plsc.store_scatter(obufs[4 * p + t], [rows, cols], row)

        def fire_out(k, p):
            blk = start + k
            for t in range(4):
                pltpu.async_copy(
                    obufs[4 * p + t], qs[t].at[pl.ds(blk * 16, 16)], osems[p])

        def step(k, p):
            @pl.when(k + 1 < nb)
            def _():
                stage_in(k + 1, 1 - p)

            wait_in(k, p)

            @pl.when(k >= 2)
            def _():
                drain_out(k - 2, p)

            extract(p)
            fire_out(k, p)

        def block_body(k, _):
            @pl.when(k % 2 == 0)
            def _():
                step(k, 0)

            @pl.when(k % 2 == 1)
            def _():
                step(k, 1)

            return ()

        stage_in(0, 0)
        lax.fori_loop(0, nb, block_body, ())
        @pl.when(nb >= 2)
        def _():
            @pl.when(nb % 2 == 0)
            def _():
                drain_out(nb - 2, 0)
                drain_out(nb - 1, 1)

            @pl.when(nb % 2 == 1)
            def _():
                drain_out(nb - 2, 1)
                drain_out(nb - 1, 0)

        @pl.when(wid == 0)
        def _():
            for t in range(4):
                pltpu.sync_copy(tqs[t], tstage)
                pltpu.sync_copy(tstage, qs[t].at[pl.ds(TAIL_V // 8, 8)])

    return relayout_kernel


_relayout_sc = _make_relayout()


def _make_gather():
    mesh = plsc.VectorSubcoreMesh(core_axis_name="c", subcore_axis_name="s")

    @functools.partial(
        pl.kernel,
        mesh=mesh,
        out_type=jax.ShapeDtypeStruct((B, EMB), jnp.float32),
        scratch_types=[
            pltpu.VMEM((CHUNK,), jnp.int32),
            pltpu.VMEM((CHUNK,), jnp.int32),
            [pltpu.VMEM((CHUNK, PART), jnp.float32) for _ in range(4)],
            [pltpu.VMEM((CHUNK, PART), jnp.float32) for _ in range(4)],
            pltpu.SemaphoreType.DMA,
            pltpu.SemaphoreType.DMA,
        ],
        compiler_params=pltpu.CompilerParams(use_tc_tiling_on_sc=False),
    )
    def emb_kernel(idx_hbm, w0, w1, w2, w3, out_hbm,
                   idx_a, idx_b, bufs_a, bufs_b, sem_a, sem_b):
        wid = lax.axis_index("s") * 2 + lax.axis_index("c")
        base = wid * B_PER_W
        ws = (w0, w1, w2, w3)

        def fetch(ci, idx_v, bufs, sem):
            row0 = base + ci * CHUNK
            pltpu.sync_copy(idx_hbm.at[pl.ds(row0, CHUNK)], idx_v)
            for t in range(4):
                pltpu.async_copy(ws[t].at[idx_v], bufs[t], sem)

        def drain_and_write(ci, idx_v, bufs, sem):
            row0 = base + ci * CHUNK
            for t in range(4):
                pltpu.make_async_copy(ws[t].at[idx_v], bufs[t], sem).wait()
            for t in range(4):
                pltpu.sync_copy(
                    bufs[t],
                    out_hbm.at[pl.ds(row0, CHUNK), pl.ds(t * PART, PART)],
                )

        def fetch_next(ci, idx_v, bufs, sem):
            @pl.when(ci + 1 < N_CHUNKS)
            def _():
                fetch(ci + 1, idx_v, bufs, sem)

        def body(ci, _):
            @pl.when(ci % 2 == 0)
            def _():
                fetch_next(ci, idx_b, bufs_b, sem_b)
                drain_and_write(ci, idx_a, bufs_a, sem_a)

            @pl.when(ci % 2 == 1)
            def _():
                fetch_next(ci, idx_a, bufs_a, sem_a)
                drain_and_write(ci, idx_b, bufs_b, sem_b)

            return ()

        fetch(0, idx_a, bufs_a, sem_a)
        lax.fori_loop(0, N_CHUNKS, body, ())

    return emb_kernel


_gather = _make_gather()


def kernel(x, W0, W1, W2, W3):
    idx = x.reshape(-1).astype(jnp.int32)
    tails = [W[TAIL_V:, :].reshape(8, 128) for W in (W0, W1, W2, W3)]
    qs = _relayout_sc(W0.T, W1.T, W2.T, W3.T, *tails)
    qs = [q.reshape(VOCAB, PART) for q in qs]
    out = _gather(idx, *qs)
    return out.reshape(BATCH, HIST, EMB)


# speedup vs baseline: 1.0102x; 1.0102x over previous
import functools

import jax
import jax.numpy as jnp
from jax import lax
from jax.experimental import pallas as pl
from jax.experimental.pallas import tpu as pltpu, tpu_sc as plsc

VOCAB = 1000000
EMB = 64
N_PART = 4
PART = EMB // N_PART
BATCH = 16384
HIST = 50
B = BATCH * HIST

NW = 32
B_PER_W = B // NW
CHUNK = 800
N_CHUNKS = B_PER_W // CHUNK

NBLK_FULL = VOCAB // 128
TAIL_V = NBLK_FULL * 128
Q_ROWS = VOCAB // 8
BLK_BASE = NBLK_FULL // NW
BLK_REM = NBLK_FULL - BLK_BASE * NW


def _make_relayout():
    mesh = plsc.VectorSubcoreMesh(core_axis_name="c", subcore_axis_name="s")
    out_t = jax.ShapeDtypeStruct((Q_ROWS, 128), jnp.float32)

    @functools.partial(
        pl.kernel,
        mesh=mesh,
        out_type=(out_t, out_t, out_t, out_t),
        scratch_types=[
            [pltpu.VMEM((16, 129), jnp.float32) for _ in range(8)],
            [pltpu.VMEM((16, 128), jnp.float32) for _ in range(8)],
            pltpu.VMEM((8, 128), jnp.float32),
            pltpu.SemaphoreType.DMA,
            pltpu.SemaphoreType.DMA,
            pltpu.SemaphoreType.DMA,
            pltpu.SemaphoreType.DMA,
        ],
        compiler_params=pltpu.CompilerParams(
            use_tc_tiling_on_sc=True, needs_layout_passes=False,
            disable_bounds_checks=True),
    )
    def relayout_kernel(t0, t1, t2, t3, tq0, tq1, tq2, tq3,
                        q0, q1, q2, q3, inbufs, obufs, tstage,
                        isem_a, isem_b, osem_a, osem_b):
        wid = lax.axis_index("s") * 2 + lax.axis_index("c")
        ts = (t0, t1, t2, t3)
        tqs = (tq0, tq1, tq2, tq3)
        qs = (q0, q1, q2, q3)
        start = wid * BLK_BASE + jnp.minimum(wid, BLK_REM)
        nb = jnp.where(wid < BLK_REM, BLK_BASE + 1, BLK_BASE)
        lanes = lax.iota(jnp.int32, 16)
        isems = (isem_a, isem_b)
        osems = (osem_a, osem_b)

        def stage_in(k, p):
            v0 = (start + k) * 128
            for t in range(4):
                pltpu.async_copy(
                    ts[t].at[:, pl.ds(v0, 128)], inbufs[4 * p + t].at[:, pl.ds(0, 128)], isems[p])

        def wait_in(k, p):
            v0 = (start + k) * 128
            for t in range(4):
                pltpu.make_async_copy(
                    ts[t].at[:, pl.ds(v0, 128)],
                    inbufs[4 * p + t].at[:, pl.ds(0, 128)], isems[p]).wait()

        def drain_out(k, p):
            blk = start + k
            for t in range(4):
                pltpu.make_async_copy(
                    obufs[4 * p + t], qs[t].at[pl.ds(blk * 16, 16)],
                    osems[p]).wait()

        def extract(p):
            @plsc.parallel_loop(0, 16, unroll=4)
            def group_body(g):
                c8 = g * 8
                rows = jnp.broadcast_to(g, (16,))
                for dc in range(8):
                    vcol = jnp.broadcast_to(c8 + dc, (16,))
                    cols = dc * PART + lanes
                    for t in range(4):
                        row = plsc.load_gather(inbufs[4 * p + t], [lanes, vcol])
                        plsc.store_scatter(obufs[4 * p + t], [rows, cols], row)

        def fire_out(k, p):
            blk = start + k
            for t in range(4):
                pltpu.async_copy(
                    obufs[4 * p + t], qs[t].at[pl.ds(blk * 16, 16)], osems[p])

        def step(k, p):
            @pl.when(k + 1 < nb)
            def _():
                stage_in(k + 1, 1 - p)

            wait_in(k, p)

            @pl.when(k >= 2)
            def _():
                drain_out(k - 2, p)

            extract(p)
            fire_out(k, p)

        def block_body(k, _):
            @pl.when(k % 2 == 0)
            def _():
                step(k, 0)

            @pl.when(k % 2 == 1)
            def _():
                step(k, 1)

            return ()

        stage_in(0, 0)
        lax.fori_loop(0, nb, block_body, ())
        @pl.when(nb >= 2)
        def _():
            @pl.when(nb % 2 == 0)
            def _():
                drain_out(nb - 2, 0)
                drain_out(nb - 1, 1)

            @pl.when(nb % 2 == 1)
            def _():
                drain_out(nb - 2, 1)
                drain_out(nb - 1, 0)

        @pl.when(wid == 0)
        def _():
            for t in range(4):
                pltpu.sync_copy(tqs[t], tstage)
                pltpu.sync_copy(tstage, qs[t].at[pl.ds(TAIL_V // 8, 8)])

    return relayout_kernel


_relayout_sc = _make_relayout()


def _make_gather():
    mesh = plsc.VectorSubcoreMesh(core_axis_name="c", subcore_axis_name="s")

    @functools.partial(
        pl.kernel,
        mesh=mesh,
        out_type=jax.ShapeDtypeStruct((B, EMB), jnp.float32),
        scratch_types=[
            pltpu.VMEM((CHUNK,), jnp.int32),
            pltpu.VMEM((CHUNK,), jnp.int32),
            [pltpu.VMEM((CHUNK, PART), jnp.float32) for _ in range(4)],
            [pltpu.VMEM((CHUNK, PART), jnp.float32) for _ in range(4)],
            pltpu.SemaphoreType.DMA,
            pltpu.SemaphoreType.DMA,
        ],
        compiler_params=pltpu.CompilerParams(use_tc_tiling_on_sc=False),
    )
    def emb_kernel(idx_hbm, w0, w1, w2, w3, out_hbm,
                   idx_a, idx_b, bufs_a, bufs_b, sem_a, sem_b):
        wid = lax.axis_index("s") * 2 + lax.axis_index("c")
        base = wid * B_PER_W
        ws = (w0, w1, w2, w3)

        def fetch(ci, idx_v, bufs, sem):
            row0 = base + ci * CHUNK
            pltpu.sync_copy(idx_hbm.at[pl.ds(row0, CHUNK)], idx_v)
            for t in range(4):
                pltpu.async_copy(ws[t].at[idx_v], bufs[t], sem)

        def drain_and_write(ci, idx_v, bufs, sem):
            row0 = base + ci * CHUNK
            for t in range(4):
                pltpu.make_async_copy(ws[t].at[idx_v], bufs[t], sem).wait()
            for t in range(4):
                pltpu.sync_copy(
                    bufs[t],
                    out_hbm.at[pl.ds(row0, CHUNK), pl.ds(t * PART, PART)],
                )

        def fetch_next(ci, idx_v, bufs, sem):
            @pl.when(ci + 1 < N_CHUNKS)
            def _():
                fetch(ci + 1, idx_v, bufs, sem)

        def body(ci, _):
            @pl.when(ci % 2 == 0)
            def _():
                fetch_next(ci, idx_b, bufs_b, sem_b)
                drain_and_write(ci, idx_a, bufs_a, sem_a)

            @pl.when(ci % 2 == 1)
            def _():
                fetch_next(ci, idx_a, bufs_a, sem_a)
                drain_and_write(ci, idx_b, bufs_b, sem_b)

            return ()

        fetch(0, idx_a, bufs_a, sem_a)
        lax.fori_loop(0, N_CHUNKS, body, ())

    return emb_kernel


_gather = _make_gather()


def kernel(x, W0, W1, W2, W3):
    idx = x.reshape(-1).astype(jnp.int32)
    tails = [W[TAIL_V:, :].reshape(8, 128) for W in (W0, W1, W2, W3)]
    qs = _relayout_sc(W0.T, W1.T, W2.T, W3.T, *tails)
    qs = [q.reshape(VOCAB, PART) for q in qs]
    out = _gather(idx, *qs)
    return out.reshape(BATCH, HIST, EMB)


# speedup vs baseline: 1.0169x; 1.0067x over previous
import functools

import jax
import jax.numpy as jnp
from jax import lax
from jax.experimental import pallas as pl
from jax.experimental.pallas import tpu as pltpu, tpu_sc as plsc

VOCAB = 1000000
EMB = 64
N_PART = 4
PART = EMB // N_PART
BATCH = 16384
HIST = 50
B = BATCH * HIST

NW = 32
B_PER_W = B // NW
CHUNK = 800
N_CHUNKS = B_PER_W // CHUNK

NBLK_FULL = VOCAB // 128
TAIL_V = NBLK_FULL * 128
Q_ROWS = VOCAB // 8
BLK_BASE = NBLK_FULL // NW
BLK_REM = NBLK_FULL - BLK_BASE * NW


def _make_relayout():
    mesh = plsc.VectorSubcoreMesh(core_axis_name="c", subcore_axis_name="s")
    out_t = jax.ShapeDtypeStruct((Q_ROWS, 128), jnp.float32)

    @functools.partial(
        pl.kernel,
        mesh=mesh,
        out_type=(out_t, out_t, out_t, out_t),
        scratch_types=[
            [pltpu.VMEM((16, 129), jnp.float32) for _ in range(8)],
            [pltpu.VMEM((16, 128), jnp.float32) for _ in range(8)],
            pltpu.VMEM((8, 128), jnp.float32),
            pltpu.SemaphoreType.DMA,
            pltpu.SemaphoreType.DMA,
            pltpu.SemaphoreType.DMA,
            pltpu.SemaphoreType.DMA,
        ],
        compiler_params=pltpu.CompilerParams(
            use_tc_tiling_on_sc=True, needs_layout_passes=False,
            disable_bounds_checks=True),
    )
    def relayout_kernel(t0, t1, t2, t3, tq0, tq1, tq2, tq3,
                        q0, q1, q2, q3, inbufs, obufs, tstage,
                        isem_a, isem_b, osem_a, osem_b):
        wid = lax.axis_index("s") * 2 + lax.axis_index("c")
        ts = (t0, t1, t2, t3)
        tqs = (tq0, tq1, tq2, tq3)
        qs = (q0, q1, q2, q3)
        start = wid * BLK_BASE + jnp.minimum(wid, BLK_REM)
        nb = jnp.where(wid < BLK_REM, BLK_BASE + 1, BLK_BASE)
        lanes = lax.iota(jnp.int32, 16)
        isems = (isem_a, isem_b)
        osems = (osem_a, osem_b)

        def stage_in(k, p):
            v0 = (start + k) * 128
            for t in range(4):
                pltpu.async_copy(
                    ts[t].at[:, pl.ds(v0, 128)], inbufs[4 * p + t].at[:, pl.ds(0, 128)], isems[p])

        def wait_in(k, p):
            v0 = (start + k) * 128
            for t in range(4):
                pltpu.make_async_copy(
                    ts[t].at[:, pl.ds(v0, 128)],
                    inbufs[4 * p + t].at[:, pl.ds(0, 128)], isems[p]).wait()

        def drain_out(k, p):
            blk = start + k
            for t in range(4):
                pltpu.make_async_copy(
                    obufs[4 * p + t], qs[t].at[pl.ds(blk * 16, 16)],
                    osems[p]).wait()

        def extract(p):
            @plsc.parallel_loop(0, 128, unroll=8)
            def col_body(c):
                rows = jnp.broadcast_to(c // 8, (16,))
                vcol = jnp.broadcast_to(c, (16,))
                cols = (c % 8) * PART + lanes
                for t in range(4):
                    row = plsc.load_gather(inbufs[4 * p + t], [lanes, vcol])
                    plsc.store_scatter(obufs[4 * p + t], [rows, cols], row)

        def fire_out(k, p):
            blk = start + k
            for t in range(4):
                pltpu.async_copy(
                    obufs[4 * p + t], qs[t].at[pl.ds(blk * 16, 16)], osems[p])

        def step(k, p):
            @pl.when(k + 1 < nb)
            def _():
                stage_in(k + 1, 1 - p)

            wait_in(k, p)

            @pl.when(k >= 2)
            def _():
                drain_out(k - 2, p)

            extract(p)
            fire_out(k, p)

        def block_body(k, _):
            @pl.when(k % 2 == 0)
            def _():
                step(k, 0)

            @pl.when(k % 2 == 1)
            def _():
                step(k, 1)

            return ()

        stage_in(0, 0)
        lax.fori_loop(0, nb, block_body, ())
        @pl.when(nb >= 2)
        def _():
            @pl.when(nb % 2 == 0)
            def _():
                drain_out(nb - 2, 0)
                drain_out(nb - 1, 1)

            @pl.when(nb % 2 == 1)
            def _():
                drain_out(nb - 2, 1)
                drain_out(nb - 1, 0)

        @pl.when(wid == 0)
        def _():
            for t in range(4):
                pltpu.sync_copy(tqs[t], tstage)
                pltpu.sync_copy(tstage, qs[t].at[pl.ds(TAIL_V // 8, 8)])

    return relayout_kernel


_relayout_sc = _make_relayout()


def _make_gather():
    mesh = plsc.VectorSubcoreMesh(core_axis_name="c", subcore_axis_name="s")

    @functools.partial(
        pl.kernel,
        mesh=mesh,
        out_type=jax.ShapeDtypeStruct((B, EMB), jnp.float32),
        scratch_types=[
            pltpu.VMEM((CHUNK,), jnp.int32),
            pltpu.VMEM((CHUNK,), jnp.int32),
            [pltpu.VMEM((CHUNK, PART), jnp.float32) for _ in range(4)],
            [pltpu.VMEM((CHUNK, PART), jnp.float32) for _ in range(4)],
            pltpu.SemaphoreType.DMA,
            pltpu.SemaphoreType.DMA,
        ],
        compiler_params=pltpu.CompilerParams(use_tc_tiling_on_sc=False),
    )
    def emb_kernel(idx_hbm, w0, w1, w2, w3, out_hbm,
                   idx_a, idx_b, bufs_a, bufs_b, sem_a, sem_b):
        wid = lax.axis_index("s") * 2 + lax.axis_index("c")
        base = wid * B_PER_W
        ws = (w0, w1, w2, w3)

        def fetch(ci, idx_v, bufs, sem):
            row0 = base + ci * CHUNK
            pltpu.sync_copy(idx_hbm.at[pl.ds(row0, CHUNK)], idx_v)
            for t in range(4):
                pltpu.async_copy(ws[t].at[idx_v], bufs[t], sem)

        def drain_and_write(ci, idx_v, bufs, sem):
            row0 = base + ci * CHUNK
            for t in range(4):
                pltpu.make_async_copy(ws[t].at[idx_v], bufs[t], sem).wait()
            for t in range(4):
                pltpu.sync_copy(
                    bufs[t],
                    out_hbm.at[pl.ds(row0, CHUNK), pl.ds(t * PART, PART)],
                )

        def fetch_next(ci, idx_v, bufs, sem):
            @pl.when(ci + 1 < N_CHUNKS)
            def _():
                fetch(ci + 1, idx_v, bufs, sem)

        def body(ci, _):
            @pl.when(ci % 2 == 0)
            def _():
                fetch_next(ci, idx_b, bufs_b, sem_b)
                drain_and_write(ci, idx_a, bufs_a, sem_a)

            @pl.when(ci % 2 == 1)
            def _():
                fetch_next(ci, idx_a, bufs_a, sem_a)
                drain_and_write(ci, idx_b, bufs_b, sem_b)

            return ()

        fetch(0, idx_a, bufs_a, sem_a)
        lax.fori_loop(0, N_CHUNKS, body, ())

    return emb_kernel


_gather = _make_gather()


def kernel(x, W0, W1, W2, W3):
    idx = x.reshape(-1).astype(jnp.int32)
    tails = [W[TAIL_V:, :].reshape(8, 128) for W in (W0, W1, W2, W3)]
    qs = _relayout_sc(W0.T, W1.T, W2.T, W3.T, *tails)
    qs = [q.reshape(VOCAB, PART) for q in qs]
    out = _gather(idx, *qs)
    return out.reshape(BATCH, HIST, EMB)


# speedup vs baseline: 1.0169x; 1.0000x over previous
import functools

import jax
import jax.numpy as jnp
from jax import lax
from jax.experimental import pallas as pl
from jax.experimental.pallas import tpu as pltpu, tpu_sc as plsc

VOCAB = 1000000
EMB = 64
N_PART = 4
PART = EMB // N_PART
BATCH = 16384
HIST = 50
B = BATCH * HIST

NW = 32
B_PER_W = B // NW
CHUNK = 800
N_CHUNKS = B_PER_W // CHUNK

NBLK_FULL = VOCAB // 128
TAIL_V = NBLK_FULL * 128
Q_ROWS = VOCAB // 8
BLK_BASE = NBLK_FULL // NW
BLK_REM = NBLK_FULL - BLK_BASE * NW


def _make_relayout():
    mesh = plsc.VectorSubcoreMesh(core_axis_name="c", subcore_axis_name="s")
    out_t = jax.ShapeDtypeStruct((Q_ROWS, 128), jnp.float32)

    @functools.partial(
        pl.kernel,
        mesh=mesh,
        out_type=(out_t, out_t, out_t, out_t),
        scratch_types=[
            [pltpu.VMEM((16, 129), jnp.float32) for _ in range(8)],
            [pltpu.VMEM((16, 128), jnp.float32) for _ in range(8)],
            pltpu.VMEM((8, 128), jnp.float32),
            pltpu.SemaphoreType.DMA,
            pltpu.SemaphoreType.DMA,
            pltpu.SemaphoreType.DMA,
            pltpu.SemaphoreType.DMA,
        ],
        compiler_params=pltpu.CompilerParams(
            use_tc_tiling_on_sc=True, needs_layout_passes=False,
            disable_bounds_checks=True),
    )
    def relayout_kernel(t0, t1, t2, t3, tq0, tq1, tq2, tq3,
                        q0, q1, q2, q3, inbufs, obufs, tstage,
                        isem_a, isem_b, osem_a, osem_b):
        wid = lax.axis_index("s") * 2 + lax.axis_index("c")
        ts = (t0, t1, t2, t3)
        tqs = (tq0, tq1, tq2, tq3)
        qs = (q0, q1, q2, q3)
        start = wid * BLK_BASE + jnp.minimum(wid, BLK_REM)
        nb = jnp.where(wid < BLK_REM, BLK_BASE + 1, BLK_BASE)
        lanes = lax.iota(jnp.int32, 16)
        isems = (isem_a, isem_b)
        osems = (osem_a, osem_b)

        def stage_in(k, p):
            v0 = (start + k) * 128
            for t in range(4):
                pltpu.async_copy(
                    ts[t].at[:, pl.ds(v0, 128)], inbufs[4 * p + t].at[:, pl.ds(0, 128)], isems[p])

        def wait_in(k, p):
            v0 = (start + k) * 128
            for t in range(4):
                pltpu.make_async_copy(
                    ts[t].at[:, pl.ds(v0, 128)],
                    inbufs[4 * p + t].at[:, pl.ds(0, 128)], isems[p]).wait()

        def drain_out(k, p):
            blk = start + k
            for t in range(4):
                pltpu.make_async_copy(
                    obufs[4 * p + t], qs[t].at[pl.ds(blk * 16, 16)],
                    osems[p]).wait()

        def extract(p):
            @plsc.parallel_loop(0, 128, unroll=16)
            def col_body(c):
                rows = jnp.broadcast_to(c // 8, (16,))
                vcol = jnp.broadcast_to(c, (16,))
                cols = (c % 8) * PART + lanes
                for t in range(4):
                    row = plsc.load_gather(inbufs[4 * p + t], [lanes, vcol])
                    plsc.store_scatter(obufs[4 * p + t], [rows, cols], row)

        def fire_out(k, p):
            blk = start + k
            for t in range(4):
                pltpu.async_copy(
                    obufs[4 * p + t], qs[t].at[pl.ds(blk * 16, 16)], osems[p])

        def step(k, p):
            @pl.when(k + 1 < nb)
            def _():
                stage_in(k + 1, 1 - p)

            wait_in(k, p)

            @pl.when(k >= 2)
            def _():
                drain_out(k - 2, p)

            extract(p)
            fire_out(k, p)

        def block_body(k, _):
            @pl.when(k % 2 == 0)
            def _():
                step(k, 0)

            @pl.when(k % 2 == 1)
            def _():
                step(k, 1)

            return ()

        stage_in(0, 0)
        lax.fori_loop(0, nb, block_body, ())
        @pl.when(nb >= 2)
        def _():
            @pl.when(nb % 2 == 0)
            def _():
                drain_out(nb - 2, 0)
                drain_out(nb - 1, 1)

            @pl.when(nb % 2 == 1)
            def _():
                drain_out(nb - 2, 1)
                drain_out(nb - 1, 0)

        @pl.when(wid == 0)
        def _():
            for t in range(4):
                pltpu.sync_copy(tqs[t], tstage)
                pltpu.sync_copy(tstage, qs[t].at[pl.ds(TAIL_V // 8, 8)])

    return relayout_kernel


_relayout_sc = _make_relayout()


def _make_gather():
    mesh = plsc.VectorSubcoreMesh(core_axis_name="c", subcore_axis_name="s")

    @functools.partial(
        pl.kernel,
        mesh=mesh,
        out_type=jax.ShapeDtypeStruct((B, EMB), jnp.float32),
        scratch_types=[
            pltpu.VMEM((CHUNK,), jnp.int32),
            pltpu.VMEM((CHUNK,), jnp.int32),
            [pltpu.VMEM((CHUNK, PART), jnp.float32) for _ in range(4)],
            [pltpu.VMEM((CHUNK, PART), jnp.float32) for _ in range(4)],
            pltpu.SemaphoreType.DMA,
            pltpu.SemaphoreType.DMA,
        ],
        compiler_params=pltpu.CompilerParams(use_tc_tiling_on_sc=False),
    )
    def emb_kernel(idx_hbm, w0, w1, w2, w3, out_hbm,
                   idx_a, idx_b, bufs_a, bufs_b, sem_a, sem_b):
        wid = lax.axis_index("s") * 2 + lax.axis_index("c")
        base = wid * B_PER_W
        ws = (w0, w1, w2, w3)

        def fetch(ci, idx_v, bufs, sem):
            row0 = base + ci * CHUNK
            pltpu.sync_copy(idx_hbm.at[pl.ds(row0, CHUNK)], idx_v)
            for t in range(4):
                pltpu.async_copy(ws[t].at[idx_v], bufs[t], sem)

        def drain_and_write(ci, idx_v, bufs, sem):
            row0 = base + ci * CHUNK
            for t in range(4):
                pltpu.make_async_copy(ws[t].at[idx_v], bufs[t], sem).wait()
            for t in range(4):
                pltpu.sync_copy(
                    bufs[t],
                    out_hbm.at[pl.ds(row0, CHUNK), pl.ds(t * PART, PART)],
                )

        def fetch_next(ci, idx_v, bufs, sem):
            @pl.when(ci + 1 < N_CHUNKS)
            def _():
                fetch(ci + 1, idx_v, bufs, sem)

        def body(ci, _):
            @pl.when(ci % 2 == 0)
            def _():
                fetch_next(ci, idx_b, bufs_b, sem_b)
                drain_and_write(ci, idx_a, bufs_a, sem_a)

            @pl.when(ci % 2 == 1)
            def _():
                fetch_next(ci, idx_a, bufs_a, sem_a)
                drain_and_write(ci, idx_b, bufs_b, sem_b)

            return ()

        fetch(0, idx_a, bufs_a, sem_a)
        lax.fori_loop(0, N_CHUNKS, body, ())

    return emb_kernel


_gather = _make_gather()


def kernel(x, W0, W1, W2, W3):
    idx = x.reshape(-1).astype(jnp.int32)
    tails = [W[TAIL_V:, :].reshape(8, 128) for W in (W0, W1, W2, W3)]
    qs = _relayout_sc(W0.T, W1.T, W2.T, W3.T, *tails)
    qs = [q.reshape(VOCAB, PART) for q in qs]
    out = _gather(idx, *qs)
    return out.reshape(BATCH, HIST, EMB)


# speedup vs baseline: 1.0177x; 1.0007x over previous
import functools

import jax
import jax.numpy as jnp
from jax import lax
from jax.experimental import pallas as pl
from jax.experimental.pallas import tpu as pltpu, tpu_sc as plsc

VOCAB = 1000000
EMB = 64
N_PART = 4
PART = EMB // N_PART
BATCH = 16384
HIST = 50
B = BATCH * HIST

NW = 32
B_PER_W = B // NW
CHUNK = 800
N_CHUNKS = B_PER_W // CHUNK

NBLK_FULL = VOCAB // 128
TAIL_V = NBLK_FULL * 128
Q_ROWS = VOCAB // 8
BLK_BASE = NBLK_FULL // NW
BLK_REM = NBLK_FULL - BLK_BASE * NW


def _make_relayout():
    mesh = plsc.VectorSubcoreMesh(core_axis_name="c", subcore_axis_name="s")
    out_t = jax.ShapeDtypeStruct((Q_ROWS, 128), jnp.float32)

    @functools.partial(
        pl.kernel,
        mesh=mesh,
        out_type=(out_t, out_t, out_t, out_t),
        scratch_types=[
            [pltpu.VMEM((16, 129), jnp.float32) for _ in range(8)],
            [pltpu.VMEM((16, 128), jnp.float32) for _ in range(8)],
            pltpu.VMEM((8, 128), jnp.float32),
            pltpu.SemaphoreType.DMA,
            pltpu.SemaphoreType.DMA,
            pltpu.SemaphoreType.DMA,
            pltpu.SemaphoreType.DMA,
        ],
        compiler_params=pltpu.CompilerParams(
            use_tc_tiling_on_sc=True, needs_layout_passes=False,
            disable_bounds_checks=True),
    )
    def relayout_kernel(t0, t1, t2, t3, tq0, tq1, tq2, tq3,
                        q0, q1, q2, q3, inbufs, obufs, tstage,
                        isem_a, isem_b, osem_a, osem_b):
        wid = lax.axis_index("s") * 2 + lax.axis_index("c")
        ts = (t0, t1, t2, t3)
        tqs = (tq0, tq1, tq2, tq3)
        qs = (q0, q1, q2, q3)
        start = wid * BLK_BASE + jnp.minimum(wid, BLK_REM)
        nb = jnp.where(wid < BLK_REM, BLK_BASE + 1, BLK_BASE)
        lanes = lax.iota(jnp.int32, 16)
        isems = (isem_a, isem_b)
        osems = (osem_a, osem_b)

        def stage_in(k, p):
            v0 = (start + k) * 128
            for t in range(4):
                pltpu.async_copy(
                    ts[t].at[:, pl.ds(v0, 128)], inbufs[4 * p + t].at[:, pl.ds(0, 128)], isems[p])

        def wait_in(k, p):
            v0 = (start + k) * 128
            for t in range(4):
                pltpu.make_async_copy(
                    ts[t].at[:, pl.ds(v0, 128)],
                    inbufs[4 * p + t].at[:, pl.ds(0, 128)], isems[p]).wait()

        def drain_out(k, p):
            blk = start + k
            for t in range(4):
                pltpu.make_async_copy(
                    obufs[4 * p + t], qs[t].at[pl.ds(blk * 16, 16)],
                    osems[p]).wait()

        def extract(p):
            @plsc.parallel_loop(0, 128, unroll=16)
            def col_body(c):
                vcol = jnp.broadcast_to(c, (16,))
                g = c // 8
                c0 = (c % 8) * PART
                for t in range(4):
                    row = plsc.load_gather(inbufs[4 * p + t], [lanes, vcol])
                    obufs[4 * p + t][g, pl.ds(c0, PART)] = row

        def fire_out(k, p):
            blk = start + k
            for t in range(4):
                pltpu.async_copy(
                    obufs[4 * p + t], qs[t].at[pl.ds(blk * 16, 16)], osems[p])

        def step(k, p):
            @pl.when(k + 1 < nb)
            def _():
                stage_in(k + 1, 1 - p)

            wait_in(k, p)

            @pl.when(k >= 2)
            def _():
                drain_out(k - 2, p)

            extract(p)
            fire_out(k, p)

        def block_body(k, _):
            @pl.when(k % 2 == 0)
            def _():
                step(k, 0)

            @pl.when(k % 2 == 1)
            def _():
                step(k, 1)

            return ()

        stage_in(0, 0)
        lax.fori_loop(0, nb, block_body, ())
        @pl.when(nb >= 2)
        def _():
            @pl.when(nb % 2 == 0)
            def _():
                drain_out(nb - 2, 0)
                drain_out(nb - 1, 1)

            @pl.when(nb % 2 == 1)
            def _():
                drain_out(nb - 2, 1)
                drain_out(nb - 1, 0)

        @pl.when(wid == 0)
        def _():
            for t in range(4):
                pltpu.sync_copy(tqs[t], tstage)
                pltpu.sync_copy(tstage, qs[t].at[pl.ds(TAIL_V // 8, 8)])

    return relayout_kernel


_relayout_sc = _make_relayout()


def _make_gather():
    mesh = plsc.VectorSubcoreMesh(core_axis_name="c", subcore_axis_name="s")

    @functools.partial(
        pl.kernel,
        mesh=mesh,
        out_type=jax.ShapeDtypeStruct((B, EMB), jnp.float32),
        scratch_types=[
            pltpu.VMEM((CHUNK,), jnp.int32),
            pltpu.VMEM((CHUNK,), jnp.int32),
            [pltpu.VMEM((CHUNK, PART), jnp.float32) for _ in range(4)],
            [pltpu.VMEM((CHUNK, PART), jnp.float32) for _ in range(4)],
            pltpu.SemaphoreType.DMA,
            pltpu.SemaphoreType.DMA,
        ],
        compiler_params=pltpu.CompilerParams(use_tc_tiling_on_sc=False),
    )
    def emb_kernel(idx_hbm, w0, w1, w2, w3, out_hbm,
                   idx_a, idx_b, bufs_a, bufs_b, sem_a, sem_b):
        wid = lax.axis_index("s") * 2 + lax.axis_index("c")
        base = wid * B_PER_W
        ws = (w0, w1, w2, w3)

        def fetch(ci, idx_v, bufs, sem):
            row0 = base + ci * CHUNK
            pltpu.sync_copy(idx_hbm.at[pl.ds(row0, CHUNK)], idx_v)
            for t in range(4):
                pltpu.async_copy(ws[t].at[idx_v], bufs[t], sem)

        def drain_and_write(ci, idx_v, bufs, sem):
            row0 = base + ci * CHUNK
            for t in range(4):
                pltpu.make_async_copy(ws[t].at[idx_v], bufs[t], sem).wait()
            for t in range(4):
                pltpu.sync_copy(
                    bufs[t],
                    out_hbm.at[pl.ds(row0, CHUNK), pl.ds(t * PART, PART)],
                )

        def fetch_next(ci, idx_v, bufs, sem):
            @pl.when(ci + 1 < N_CHUNKS)
            def _():
                fetch(ci + 1, idx_v, bufs, sem)

        def body(ci, _):
            @pl.when(ci % 2 == 0)
            def _():
                fetch_next(ci, idx_b, bufs_b, sem_b)
                drain_and_write(ci, idx_a, bufs_a, sem_a)

            @pl.when(ci % 2 == 1)
            def _():
                fetch_next(ci, idx_a, bufs_a, sem_a)
                drain_and_write(ci, idx_b, bufs_b, sem_b)

            return ()

        fetch(0, idx_a, bufs_a, sem_a)
        lax.fori_loop(0, N_CHUNKS, body, ())

    return emb_kernel


_gather = _make_gather()


def kernel(x, W0, W1, W2, W3):
    idx = x.reshape(-1).astype(jnp.int32)
    tails = [W[TAIL_V:, :].reshape(8, 128) for W in (W0, W1, W2, W3)]
    qs = _relayout_sc(W0.T, W1.T, W2.T, W3.T, *tails)
    qs = [q.reshape(VOCAB, PART) for q in qs]
    out = _gather(idx, *qs)
    return out.reshape(BATCH, HIST, EMB)
